# baseline (speedup 1.0000x reference)
import functools

import jax
import jax.numpy as jnp
from jax import lax
from jax.experimental import pallas as pl
from jax.experimental.pallas import tpu as pltpu
from jax.experimental.pallas import tpu_sc as plsc

B = 16384
V = 1000000
D = 32
PK = 128 // D
VP = V // PK
NC = 2
NS = 16
L = 16
NW = NC * NS
BPW = B // NW
NG = BPW // L

_mesh = plsc.VectorSubcoreMesh(
    core_axis_name="c", subcore_axis_name="s", num_cores=NC, num_subcores=NS
)


@functools.partial(
    pl.kernel,
    out_type=(
        jax.ShapeDtypeStruct((B,), jnp.float32),
        jax.ShapeDtypeStruct((B,), jnp.float32),
        jax.ShapeDtypeStruct((B * D,), jnp.float32),
        jax.ShapeDtypeStruct((B * D,), jnp.float32),
    ),
    mesh=_mesh,
    scratch_types=[
        pltpu.VMEM((BPW,), jnp.int32),
        pltpu.VMEM((BPW,), jnp.int32),
        pltpu.VMEM((BPW // 2, 128), jnp.float32),
        pltpu.VMEM((BPW * D,), jnp.float32),
        pltpu.VMEM((BPW * D,), jnp.float32),
        pltpu.VMEM((BPW * D,), jnp.float32),
        pltpu.VMEM((BPW * L,), jnp.float32),
        pltpu.VMEM((BPW,), jnp.float32),
        pltpu.SemaphoreType.DMA,
    ],
    compiler_params=pltpu.CompilerParams(
        needs_layout_passes=False, use_tc_tiling_on_sc=True
    ),
)
def _sc_gather_score(si1, pi1, oi1, si2, pi2, oi2, si3, oi3,
                     E1p, E2p, R1p, R2p,
                     l1_out, l2_out, s3_out, o3_out,
                     iv, kv, w_v, s_v, p_v, o_v, q_v, l_v, sem):
    wid = lax.axis_index("s") * NC + lax.axis_index("c")
    base = wid * BPW

    def gather(idx_hbm, table, rows_v):
        pltpu.sync_copy(idx_hbm.at[pl.ds(base, BPW)], iv)

        def kbody(g, c):
            r0 = pl.multiple_of(g * L, L)
            kv[pl.ds(r0, L)] = iv[pl.ds(r0, L)] >> 2
            return c
        lax.fori_loop(0, NG, kbody, 0)

        for ch in range(2):
            c0 = ch * (BPW // 2)
            pltpu.async_copy(
                table.at[kv.at[pl.ds(c0, BPW // 2)]], w_v, sem
            ).wait()

            def xbody(g, c):
                r0 = pl.multiple_of(g * L, L)
                sv = iv[pl.ds(c0 + r0, L)]
                for i in range(L):
                    off = pl.multiple_of((sv[i] & 3) * D, 8)
                    rb = pl.multiple_of((c0 + r0 + i) * D, L)
                    rows_v[pl.ds(rb, L)] = w_v[r0 + i, pl.ds(off, L)]
                    rows_v[pl.ds(rb + L, L)] = w_v[r0 + i, pl.ds(off + L, L)]
                return c
            lax.fori_loop(0, NG // 2, xbody, 0)

    def distmult(l_hbm):
        def group_body(g, carry):
            row0 = pl.multiple_of(g * L, L)
            for i in range(L):
                r = row0 + i
                rb = pl.multiple_of(r * D, L)
                lo = (s_v[pl.ds(rb, L)] * p_v[pl.ds(rb, L)]
                      * o_v[pl.ds(rb, L)])
                hi = (s_v[pl.ds(rb + L, L)] * p_v[pl.ds(rb + L, L)]
                      * o_v[pl.ds(rb + L, L)])
                q_v[pl.ds(r * L, L)] = lo + hi
            rows16 = (row0 + lax.iota(jnp.int32, L)) * L
            acc = jnp.zeros((L,), jnp.float32)
            for c in range(L):
                acc = acc + plsc.load_gather(q_v, [rows16 + c])
            l_v[pl.ds(row0, L)] = 1.0 / (1.0 + jnp.exp(-acc))
            return carry
        lax.fori_loop(0, NG, group_body, 0)
        pltpu.sync_copy(l_v, l_hbm.at[pl.ds(base, BPW)])

    gather(si1, E1p, s_v)
    gather(pi1, R1p, p_v)
    gather(oi1, E1p, o_v)
    distmult(l1_out)

    gather(si2, E2p, s_v)
    gather(pi2, R2p, p_v)
    gather(oi2, E2p, o_v)
    distmult(l2_out)

    gather(si3, E1p, s_v)
    pltpu.sync_copy(s_v, s3_out.at[pl.ds(base * D, BPW * D)])
    gather(oi3, E2p, o_v)
    pltpu.sync_copy(o_v, o3_out.at[pl.ds(base * D, BPW * D)])


def _mlp_body(s3_ref, o3_ref, w1a_ref, w1b_ref, b1_ref, w2_ref, b2_ref,
              out_ref):
    h = (
        jnp.dot(s3_ref[:], w1a_ref[:], preferred_element_type=jnp.float32)
        + jnp.dot(o3_ref[:], w1b_ref[:], preferred_element_type=jnp.float32)
        + b1_ref[:]
    )
    h = jnp.maximum(h, 0.0)
    z = jnp.sum(h * w2_ref[:], axis=1) + b2_ref[0]
    out_ref[:] = 1.0 / (1.0 + jnp.exp(-z))


_mlp = pl.pallas_call(
    _mlp_body,
    out_shape=jax.ShapeDtypeStruct((B,), jnp.float32),
)


def kernel(t1, t2, t3, E1, E2, R1, R2, W1, b1, gamma, beta, mov_mean,
           mov_var, W2, b2):
    si1 = t1[:, 0].astype(jnp.int32)
    pi1 = t1[:, 1].astype(jnp.int32)
    oi1 = t1[:, 2].astype(jnp.int32)
    si2 = t2[:, 0].astype(jnp.int32)
    pi2 = t2[:, 1].astype(jnp.int32)
    oi2 = t2[:, 2].astype(jnp.int32)
    si3 = t3[:, 0].astype(jnp.int32)
    oi3 = t3[:, 2].astype(jnp.int32)

    def pack(t):
        return jnp.concatenate([t[j::PK] for j in range(PK)], axis=1)

    l1, l2, s3, o3 = _sc_gather_score(si1, pi1, oi1, si2, pi2, oi2, si3, oi3,
                                      pack(E1), pack(E2), pack(R1), pack(R2))

    scale = gamma / jnp.sqrt(mov_var + 1e-3)
    w1_eff = W1 * scale[None, :]
    b1_eff = (b1 - mov_mean) * scale + beta
    x = _mlp(s3.reshape(B, D), o3.reshape(B, D),
             w1_eff[:D], w1_eff[D:], b1_eff, W2.reshape(1, D), b2)
    return (l1, l2, x)


# speedup vs baseline: 10.5749x; 10.5749x over previous
import functools

import jax
import jax.numpy as jnp
from jax import lax
from jax.experimental import pallas as pl
from jax.experimental.pallas import tpu as pltpu
from jax.experimental.pallas import tpu_sc as plsc

B = 16384
D = 32
NC = 2
NS = 16
L = 16
NW = NC * NS
BPW = B // NW
NG = BPW // L

_mesh = plsc.VectorSubcoreMesh(
    core_axis_name="c", subcore_axis_name="s", num_cores=NC, num_subcores=NS
)


def _sigmoid_v(x):
    return 1.0 / (1.0 + jnp.exp(-x))


@functools.partial(
    pl.kernel,
    out_type=(
        jax.ShapeDtypeStruct((B,), jnp.float32),
        jax.ShapeDtypeStruct((B,), jnp.float32),
        jax.ShapeDtypeStruct((B, D), jnp.float32),
        jax.ShapeDtypeStruct((B, D), jnp.float32),
    ),
    mesh=_mesh,
    scratch_types=[
        pltpu.VMEM((BPW,), jnp.int32),
        pltpu.VMEM((BPW, D), jnp.float32),
        pltpu.VMEM((BPW, D), jnp.float32),
        pltpu.VMEM((BPW, D), jnp.float32),
        pltpu.VMEM((BPW * L,), jnp.float32),
        pltpu.VMEM((BPW,), jnp.float32),
        pltpu.SemaphoreType.DMA,
    ],
    compiler_params=pltpu.CompilerParams(
        needs_layout_passes=False, use_tc_tiling_on_sc=False
    ),
)
def _sc_gather_score(si1, pi1, oi1, si2, pi2, oi2, si3, oi3,
                     E1, E2, R1, R2,
                     l1_out, l2_out, s3_out, o3_out,
                     idx_v, s_v, p_v, o_v, q_v, l_v, sem):
    wid = lax.axis_index("s") * NC + lax.axis_index("c")
    base = wid * BPW

    def gather(idx_hbm, table, rows_v):
        pltpu.sync_copy(idx_hbm.at[pl.ds(base, BPW)], idx_v)
        pltpu.async_copy(table.at[idx_v], rows_v, sem).wait()

    def distmult(l_hbm):
        def group_body(g, carry):
            row0 = pl.multiple_of(g * L, L)
            for i in range(L):
                r = row0 + i
                lo = (s_v[r, pl.ds(0, L)] * p_v[r, pl.ds(0, L)]
                      * o_v[r, pl.ds(0, L)])
                hi = (s_v[r, pl.ds(L, L)] * p_v[r, pl.ds(L, L)]
                      * o_v[r, pl.ds(L, L)])
                q_v[pl.ds(r * L, L)] = lo + hi
            rows16 = (row0 + lax.iota(jnp.int32, L)) * L
            acc = jnp.zeros((L,), jnp.float32)
            for c in range(L):
                acc = acc + plsc.load_gather(q_v, [rows16 + c])
            l_v[pl.ds(row0, L)] = _sigmoid_v(acc)
            return carry
        lax.fori_loop(0, NG, group_body, 0)
        pltpu.sync_copy(l_v, l_hbm.at[pl.ds(base, BPW)])

    gather(si1, E1, s_v)
    gather(pi1, R1, p_v)
    gather(oi1, E1, o_v)
    distmult(l1_out)

    gather(si2, E2, s_v)
    gather(pi2, R2, p_v)
    gather(oi2, E2, o_v)
    distmult(l2_out)

    gather(si3, E1, s_v)
    pltpu.sync_copy(s_v, s3_out.at[pl.ds(base, BPW)])
    gather(oi3, E2, o_v)
    pltpu.sync_copy(o_v, o3_out.at[pl.ds(base, BPW)])


def _mlp_body(s3_ref, o3_ref, w1a_ref, w1b_ref, b1_ref, w2_ref, b2_ref,
              out_ref):
    h = (
        jnp.dot(s3_ref[:], w1a_ref[:], preferred_element_type=jnp.float32)
        + jnp.dot(o3_ref[:], w1b_ref[:], preferred_element_type=jnp.float32)
        + b1_ref[:]
    )
    h = jnp.maximum(h, 0.0)
    z = jnp.sum(h * w2_ref[:], axis=1) + b2_ref[0]
    out_ref[:] = 1.0 / (1.0 + jnp.exp(-z))


_mlp = pl.pallas_call(
    _mlp_body,
    out_shape=jax.ShapeDtypeStruct((B,), jnp.float32),
)


def kernel(t1, t2, t3, E1, E2, R1, R2, W1, b1, gamma, beta, mov_mean,
           mov_var, W2, b2):
    si1 = t1[:, 0].astype(jnp.int32)
    pi1 = t1[:, 1].astype(jnp.int32)
    oi1 = t1[:, 2].astype(jnp.int32)
    si2 = t2[:, 0].astype(jnp.int32)
    pi2 = t2[:, 1].astype(jnp.int32)
    oi2 = t2[:, 2].astype(jnp.int32)
    si3 = t3[:, 0].astype(jnp.int32)
    oi3 = t3[:, 2].astype(jnp.int32)

    l1, l2, s3, o3 = _sc_gather_score(si1, pi1, oi1, si2, pi2, oi2, si3, oi3,
                                      E1, E2, R1, R2)

    scale = gamma / jnp.sqrt(mov_var + 1e-3)
    w1_eff = W1 * scale[None, :]
    b1_eff = (b1 - mov_mean) * scale + beta
    x = _mlp(s3, o3, w1_eff[:D], w1_eff[D:], b1_eff, W2.reshape(1, D), b2)
    return (l1, l2, x)
